# baseline (speedup 1.0000x reference)
import jax
import jax.numpy as jnp
from jax.experimental import pallas as pl
from jax.experimental.pallas import tpu as pltpu


def kernel(flat, cu_seqlens):
    B = cu_seqlens.shape[0] - 1
    D = flat.shape[1]
    H = B // 2

    def body(cu_ref, flat_ref, out_ref, idx_smem, rows_v, sem_a, sem_b, sem_w):
        idx_copy = pltpu.make_async_copy(cu_ref, idx_smem, sem_a)
        idx_copy.start()
        idx_copy.wait()
        reads = [
            pltpu.make_async_copy(
                flat_ref.at[pl.ds(idx_smem[i], 1), :],
                rows_v.at[pl.ds(i, 1), :],
                sem_a if i < H else sem_b,
            )
            for i in range(B)
        ]
        for c in reads:
            c.start()
        writes = [
            pltpu.make_async_copy(
                rows_v.at[pl.ds(h * H, H), :],
                out_ref.at[pl.ds(h * H, H), :],
                sem_w,
            )
            for h in range(2)
        ]
        for c in reads[:H]:
            c.wait()
        writes[0].start()
        for c in reads[H:]:
            c.wait()
        writes[1].start()
        writes[0].wait()
        writes[1].wait()

    return pl.pallas_call(
        body,
        in_specs=[
            pl.BlockSpec(memory_space=pltpu.MemorySpace.HBM),
            pl.BlockSpec(memory_space=pltpu.MemorySpace.HBM),
        ],
        out_specs=pl.BlockSpec(memory_space=pltpu.MemorySpace.HBM),
        scratch_shapes=[
            pltpu.SMEM((B + 1,), jnp.int32),
            pltpu.VMEM((B, D), jnp.float32),
            pltpu.SemaphoreType.DMA,
            pltpu.SemaphoreType.DMA,
            pltpu.SemaphoreType.DMA,
        ],
        out_shape=jax.ShapeDtypeStruct((B, D), jnp.float32),
    )(cu_seqlens, flat)


# speedup vs baseline: 1.0071x; 1.0071x over previous
import jax
import jax.numpy as jnp
from jax.experimental import pallas as pl
from jax.experimental.pallas import tpu as pltpu


def kernel(flat, cu_seqlens):
    B = cu_seqlens.shape[0] - 1
    D = flat.shape[1]

    def body(idx_ref, flat_ref, out_ref, sem):
        copies = [
            pltpu.make_async_copy(
                flat_ref.at[pl.ds(idx_ref[i], 1), :],
                out_ref.at[pl.ds(i, 1), :],
                sem,
            )
            for i in range(B)
        ]
        for c in copies:
            c.start()
        for c in copies:
            c.wait()

    grid_spec = pltpu.PrefetchScalarGridSpec(
        num_scalar_prefetch=1,
        grid=(1,),
        in_specs=[pl.BlockSpec(memory_space=pltpu.MemorySpace.HBM)],
        out_specs=pl.BlockSpec((B, D), memory_space=pltpu.MemorySpace.VMEM),
        scratch_shapes=[pltpu.SemaphoreType.DMA],
    )

    return pl.pallas_call(
        body,
        grid_spec=grid_spec,
        out_shape=jax.ShapeDtypeStruct((B, D), jnp.float32),
    )(cu_seqlens, flat)


# speedup vs baseline: 1.0094x; 1.0023x over previous
import jax
import jax.numpy as jnp
from jax.experimental import pallas as pl
from jax.experimental.pallas import tpu as pltpu


def kernel(flat, cu_seqlens):
    B = cu_seqlens.shape[0] - 1
    D = flat.shape[1]

    def body(idx_ref, flat_ref, out_ref, sem):
        copies = [
            pltpu.make_async_copy(
                flat_ref.at[pl.ds(idx_ref[i], 1), :],
                out_ref.at[pl.ds(i, 1), :],
                sem,
            )
            for i in range(B)
        ]
        for c in copies:
            c.start()
        for c in copies:
            c.wait()

    return pl.pallas_call(
        body,
        in_specs=[
            pl.BlockSpec(memory_space=pltpu.MemorySpace.SMEM),
            pl.BlockSpec(memory_space=pltpu.MemorySpace.HBM),
        ],
        out_specs=pl.BlockSpec((B, D), memory_space=pltpu.MemorySpace.VMEM),
        scratch_shapes=[pltpu.SemaphoreType.DMA],
        out_shape=jax.ShapeDtypeStruct((B, D), jnp.float32),
    )(cu_seqlens, flat)
